# baseline (speedup 1.0000x reference)
import functools

import jax
import jax.numpy as jnp
from jax import lax
from jax.experimental import pallas as pl
from jax.experimental.pallas import tpu as pltpu
from jax.experimental.pallas import tpu_sc as plsc

B, N, K, R = 4, 1024, 32, 2
DIMS = [1, 25, 64, 38, 64]
NUM_CLASSES = 10
BN = B * N
NE = BN * K
NW = 32
PPT = BN // NW
EPT = PPT * K
CHK = 128
NCHK = EPT // CHK
PPC = CHK // K
P2T = B * 4 * K
P2PT = P2T // NW

_MESH = plsc.VectorSubcoreMesh(core_axis_name="c", subcore_axis_name="s")
f32 = jnp.float32
i32 = jnp.int32


def _wid():
    return lax.axis_index("s") * 2 + lax.axis_index("c")


def _rsqrt_nr(x):
    ibits = plsc.bitcast(x, i32)
    seed = plsc.bitcast(jnp.int32(0x5F3759DF) - (ibits >> 1), f32)
    y = seed
    for _ in range(3):
        y = y * (1.5 - 0.5 * x * y * y)
    return y


def _sca_body(xc_ref, yc_ref, zc_ref, x0_ref, gidx_ref, rm_ref,
              w0_ref, wg0_ref, bb_ref,
              b0_ref, b1_ref, z1_ref,
              xv, yv, zv, x0v, giv, rmv, b0v, b1v, z1v, wv):
    wid = _wid()
    base_pt = wid * PPT
    base_e = wid * EPT
    pltpu.sync_copy(xc_ref, xv)
    pltpu.sync_copy(yc_ref, yv)
    pltpu.sync_copy(zc_ref, zv)
    pltpu.sync_copy(x0_ref, x0v)
    pltpu.sync_copy(gidx_ref.at[pl.ds(base_e, EPT)], giv)
    pltpu.sync_copy(rm_ref.at[pl.ds(base_e, EPT)], rmv)
    pltpu.sync_copy(w0_ref, wv.at[pl.ds(0, 2)])
    pltpu.sync_copy(wg0_ref, wv.at[pl.ds(2, 2)])
    pltpu.sync_copy(bb_ref, wv.at[pl.ds(4, 2)])

    w00a = wv[0, 0:16]
    w00b = wv[0, 16:32]
    w01a = wv[1, 0:16]
    w01b = wv[1, 16:32]
    wg00a = wv[2, 0:16]
    wg00b = wv[2, 16:32]
    wg01a = wv[3, 0:16]
    wg01b = wv[3, 16:32]
    ba = wv[4, 0:16]
    bb = wv[4, 16:32]
    bga = wv[5, 0:16]
    bgb = wv[5, 16:32]
    inv_k = 1.0 / K

    def pt_body(p, _):
        g = base_pt + p
        gs = jnp.full((16,), g, dtype=i32)
        cnx = plsc.load_gather(xv, [gs])
        cny = plsc.load_gather(yv, [gs])
        cnz = plsc.load_gather(zv, [gs])
        a0 = jnp.zeros((16,), f32)
        a1 = jnp.zeros((16,), f32)
        for h in range(2):
            off = p * K + h * 16
            idx = giv[pl.ds(off, 16)]
            jx = plsc.load_gather(xv, [idx])
            jy = plsc.load_gather(yv, [idx])
            jz = plsc.load_gather(zv, [idx])
            xj = plsc.load_gather(x0v, [idx])
            dx = jx - cnx
            dy = jy - cny
            dz = jz - cnz
            d2 = dx * dx + dy * dy + dz * dz + 1e-12
            rm = rmv[pl.ds(off, 16)]
            b0 = jnp.exp(-d2) * rm
            dist = d2 * _rsqrt_nr(d2)
            b1 = b0 * jnp.exp(2.0 * dist - 1.0)
            b0v[pl.ds(off, 16)] = b0
            b1v[pl.ds(off, 16)] = b1
            a0 = a0 + b0 * xj
            a1 = a1 + b1 * xj
        s0 = jnp.sum(a0) * inv_k
        s1 = jnp.sum(a1) * inv_k
        msg_a = s0 * w00a + s1 * w01a + ba
        msg_b = s0 * w00b + s1 * w01b + bb
        gmsg_a = s0 * wg00a + s1 * wg01a + bga
        gmsg_b = s0 * wg00b + s1 * wg01b + bgb
        za = jnp.maximum(msg_a, 0.0) / (1.0 + jnp.exp(-gmsg_a))
        zb = jnp.maximum(msg_b, 0.0) / (1.0 + jnp.exp(-gmsg_b))
        z1v[p, 0:16] = za
        z1v[p, 16:32] = zb
        return 0

    lax.fori_loop(0, PPT, pt_body, 0)
    pltpu.sync_copy(b0v, b0_ref.at[pl.ds(base_e, EPT)])
    pltpu.sync_copy(b1v, b1_ref.at[pl.ds(base_e, EPT)])
    pltpu.sync_copy(z1v, z1_ref.at[pl.ds(base_pt, PPT)])


_sca = functools.partial(
    pl.kernel,
    mesh=_MESH,
    compiler_params=pltpu.CompilerParams(needs_layout_passes=False, use_tc_tiling_on_sc=False),
    out_type=(jax.ShapeDtypeStruct((NE,), f32),
              jax.ShapeDtypeStruct((NE,), f32),
              jax.ShapeDtypeStruct((BN, 32), f32)),
    scratch_types=[
        pltpu.VMEM((BN,), f32), pltpu.VMEM((BN,), f32),
        pltpu.VMEM((BN,), f32), pltpu.VMEM((BN,), f32),
        pltpu.VMEM((EPT,), i32), pltpu.VMEM((EPT,), f32),
        pltpu.VMEM((EPT,), f32), pltpu.VMEM((EPT,), f32),
        pltpu.VMEM((PPT, 32), f32),
        pltpu.VMEM((6, 32), f32),
    ],
)(_sca_body)


def _sc2_body(z1_ref, gidx2_ref, b0_ref, b1_ref,
              s1_ref,
              gi2v, b0v, b1v, rows0, rows1, s1v, sem0, sem1):
    wid = _wid()
    pltpu.sync_copy(gidx2_ref.at[pl.ds(wid * NCHK, NCHK)], gi2v)
    pltpu.sync_copy(b0_ref.at[pl.ds(wid * EPT, EPT)], b0v)
    pltpu.sync_copy(b1_ref.at[pl.ds(wid * EPT, EPT)], b1v)

    rows = (rows0, rows1)
    sems = (sem0, sem1)
    handles = [None, None]
    handles[0] = pltpu.async_copy(z1_ref.at[gi2v.at[0]], rows0, sem0)
    for c in range(NCHK):
        if c + 1 < NCHK:
            handles[(c + 1) % 2] = pltpu.async_copy(
                z1_ref.at[gi2v.at[c + 1]], rows[(c + 1) % 2],
                sems[(c + 1) % 2])
        handles[c % 2].wait()
        rv = rows[c % 2]

        def pt_body(pp, _):
            p = c * PPC + pp
            zero = jnp.zeros((16,), f32)

            def h_body(h, carry):
                a00, a01, a10, a11 = carry
                b0h = b0v[pl.ds(p * K + h * 16, 16)]
                b1h = b1v[pl.ds(p * K + h * 16, 16)]
                rbase = pp * K + h * 16
                for u in range(16):
                    lo = rv[rbase + u, 0:16]
                    hi = rv[rbase + u, 16:32]
                    b0s = b0h[u]
                    b1s = b1h[u]
                    a00 = a00 + lo * b0s
                    a01 = a01 + hi * b0s
                    a10 = a10 + lo * b1s
                    a11 = a11 + hi * b1s
                return (a00, a01, a10, a11)

            a00, a01, a10, a11 = lax.fori_loop(
                0, 2, h_body, (zero, zero, zero, zero))
            s1v[p, 0:16] = a00
            s1v[p, 16:32] = a01
            s1v[p, 32:48] = a10
            s1v[p, 48:64] = a11
            return 0

        lax.fori_loop(0, PPC, pt_body, 0)
    pltpu.sync_copy(s1v, s1_ref.at[pl.ds(wid * PPT, PPT)])


_sc2 = functools.partial(
    pl.kernel,
    mesh=_MESH,
    compiler_params=pltpu.CompilerParams(needs_layout_passes=False, use_tc_tiling_on_sc=False),
    out_type=jax.ShapeDtypeStruct((BN, 2 * 32), f32),
    scratch_types=[
        pltpu.VMEM((NCHK, CHK), i32),
        pltpu.VMEM((EPT,), f32), pltpu.VMEM((EPT,), f32),
        pltpu.VMEM((CHK, 32), f32), pltpu.VMEM((CHK, 32), f32),
        pltpu.VMEM((PPT, 64), f32),
        pltpu.SemaphoreType.DMA, pltpu.SemaphoreType.DMA,
    ],
)(_sc2_body)


def _sc3_body(z2_ref, gidxk_ref, b02_ref, b12_ref, pidx_ref,
              s2_ref,
              pidv, girows, b0r, b1r, zr0, zr1, s2v, sema, sem0, sem1):
    wid = _wid()
    pltpu.sync_copy(pidx_ref.at[pl.ds(wid * P2PT, P2PT)], pidv)
    pltpu.async_copy(gidxk_ref.at[pidv], girows, sema).wait()
    pltpu.async_copy(b02_ref.at[pidv], b0r, sema).wait()
    pltpu.async_copy(b12_ref.at[pidv], b1r, sema).wait()

    zrs = (zr0, zr1)
    sems = (sem0, sem1)
    handles = [None, None]
    handles[0] = pltpu.async_copy(z2_ref.at[girows.at[0]], zr0, sem0)
    for q in range(P2PT):
        if q + 1 < P2PT:
            handles[(q + 1) % 2] = pltpu.async_copy(
                z2_ref.at[girows.at[q + 1]], zrs[(q + 1) % 2],
                sems[(q + 1) % 2])
        handles[q % 2].wait()
        rv = zrs[q % 2]
        zero = jnp.zeros((16,), f32)

        def e_body(e2, carry):
            accs = list(carry)
            for u in range(2):
                e = e2 * 2 + u
                qs = jnp.full((16,), q, dtype=i32)
                es = jnp.full((16,), e, dtype=i32)
                b0s = plsc.load_gather(b0r, [qs, es])
                b1s = plsc.load_gather(b1r, [qs, es])
                for seg in range(4):
                    v = rv[e, pl.ds(seg * 16, 16)]
                    accs[seg] = accs[seg] + v * b0s
                    accs[4 + seg] = accs[4 + seg] + v * b1s
            return tuple(accs)

        accs = lax.fori_loop(0, K // 2, e_body, (zero,) * 8)
        for seg in range(8):
            s2v[q, pl.ds(seg * 16, 16)] = accs[seg]
    pltpu.sync_copy(s2v, s2_ref.at[pl.ds(wid * P2PT, P2PT)])


_sc3 = functools.partial(
    pl.kernel,
    mesh=_MESH,
    compiler_params=pltpu.CompilerParams(needs_layout_passes=False, use_tc_tiling_on_sc=False),
    out_type=jax.ShapeDtypeStruct((P2T, 2 * 64), f32),
    scratch_types=[
        pltpu.VMEM((P2PT,), i32),
        pltpu.VMEM((P2PT, K), i32),
        pltpu.VMEM((P2PT, K), f32), pltpu.VMEM((P2PT, K), f32),
        pltpu.VMEM((K, 64), f32), pltpu.VMEM((K, 64), f32),
        pltpu.VMEM((P2PT, 2 * 64), f32),
        pltpu.SemaphoreType.DMA,
        pltpu.SemaphoreType.DMA, pltpu.SemaphoreType.DMA,
    ],
)(_sc3_body)


def _tc2_body(s1_ref, w_ref, wg_ref, b_ref, bg_ref, z2_ref):
    s1 = s1_ref[...] * (1.0 / K)
    msg = jnp.dot(s1, w_ref[...], preferred_element_type=f32) + b_ref[...]
    gmsg = jnp.dot(s1, wg_ref[...], preferred_element_type=f32) + bg_ref[...]
    z2_ref[...] = jax.nn.relu(msg) * jax.nn.sigmoid(gmsg)


def _tc3_body(s2_ref, w2_ref, wg2_ref, b2_ref, bg2_ref,
              b0t_ref, b1t_ref, w3_ref, wg3_ref, b3_ref, bg3_ref,
              wf_ref, bf_ref, out_ref):
    inv_k = 1.0 / K
    s2 = s2_ref[...] * inv_k
    msg = jnp.dot(s2, w2_ref[...], preferred_element_type=f32) + b2_ref[...]
    gmsg = jnp.dot(s2, wg2_ref[...], preferred_element_type=f32) + bg2_ref[...]
    z3 = jax.nn.relu(msg) * jax.nn.sigmoid(gmsg)

    rows0 = []
    rows1 = []
    for q in range(16):
        blk = z3[q * K:(q + 1) * K, :]
        w0c = b0t_ref[:, q:q + 1]
        w1c = b1t_ref[:, q:q + 1]
        rows0.append(jnp.sum(blk * w0c, axis=0, keepdims=True))
        rows1.append(jnp.sum(blk * w1c, axis=0, keepdims=True))
    s3 = jnp.concatenate(
        [jnp.concatenate(rows0, axis=0),
         jnp.concatenate(rows1, axis=0)], axis=1) * inv_k
    msg3 = jnp.dot(s3, w3_ref[...], preferred_element_type=f32) + b3_ref[...]
    gmsg3 = jnp.dot(s3, wg3_ref[...], preferred_element_type=f32) + bg3_ref[...]
    out4 = jax.nn.relu(msg3) * jax.nn.sigmoid(gmsg3)

    ri = lax.broadcasted_iota(i32, (4, 16), 0)
    ci = lax.broadcasted_iota(i32, (4, 16), 1)
    pmat = jnp.where(ci // 4 == ri, 0.25, 0.0).astype(f32)
    pooled = jnp.dot(pmat, out4, preferred_element_type=f32)
    out_ref[...] = (jnp.dot(pooled, wf_ref[...], preferred_element_type=f32)
                    + bf_ref[...])


@jax.jit
def _forward_impl(xc, yc, zc, x0, gidx, gidx2, gidxk, rmflat,
                  w0pad, wg0pad, bbpad,
                  wcat1, wgcat1, b1r, bg1r,
                  wcat2, wgcat2, b2r, bg2r,
                  pidx, w3f, wg3f, b3r, bg3r, wf, bfr):
    bas0, bas1, z1 = _sca(xc, yc, zc, x0, gidx, rmflat,
                          w0pad, wg0pad, bbpad)

    s1 = _sc2(z1, gidx2, bas0, bas1)

    z2 = pl.pallas_call(
        _tc2_body,
        out_shape=jax.ShapeDtypeStruct((BN, 64), f32),
    )(s1, wcat1, wgcat1, b1r, bg1r)

    s2 = _sc3(z2, gidxk, bas0.reshape(BN, K), bas1.reshape(BN, K), pidx)

    b0t = bas0.reshape(B, N, K)[:, :4, :].reshape(16, K).T
    b1t = bas1.reshape(B, N, K)[:, :4, :].reshape(16, K).T

    out = pl.pallas_call(
        _tc3_body,
        out_shape=jax.ShapeDtypeStruct((B, NUM_CLASSES), f32),
    )(s2, wcat2, wgcat2, b2r, bg2r, b0t, b1t,
      w3f, wg3f, b3r, bg3r, wf, bfr)
    return out


def kernel(input, coords, neighbor, relative_mask,
           W0, Wg0, b0, bg0, W1, Wg1, b1, bg1,
           W2, Wg2, b2, bg2, W3, Wg3, b3, bg3, Wf, bf):
    xc = coords[..., 0].reshape(BN)
    yc = coords[..., 1].reshape(BN)
    zc = coords[..., 2].reshape(BN)
    x0 = input[:, 0, :].reshape(BN)
    nbr = neighbor.astype(i32)
    gidx = (nbr + (jnp.arange(B, dtype=i32) * N)[:, None, None]).reshape(NE)
    gidx2 = gidx.reshape(NE // CHK, CHK)
    gidxk = gidx.reshape(BN, K)
    pidx = gidx.reshape(B, N, K)[:, :4, :].reshape(P2T)
    rmflat = relative_mask.reshape(NE)

    pad7 = jnp.zeros((R, 32 - DIMS[1]), dtype=f32)
    w0pad = jnp.concatenate([W0[:, 0, :], pad7], axis=1)
    wg0pad = jnp.concatenate([Wg0[:, 0, :], pad7], axis=1)
    bbpad = jnp.concatenate(
        [jnp.stack([b0, bg0]), pad7], axis=1)

    z64 = jnp.zeros((64, 64), dtype=f32)
    wcat1 = z64.at[0:25, :].set(W1[0]).at[32:57, :].set(W1[1])
    wgcat1 = z64.at[0:25, :].set(Wg1[0]).at[32:57, :].set(Wg1[1])
    wcat2 = jnp.concatenate([W2[0], W2[1]], axis=0)
    wgcat2 = jnp.concatenate([Wg2[0], Wg2[1]], axis=0)
    w3f = jnp.concatenate([W3[0], W3[1]], axis=0)
    wg3f = jnp.concatenate([Wg3[0], Wg3[1]], axis=0)

    return _forward_impl(
        xc, yc, zc, x0, gidx, gidx2, gidxk, rmflat,
        w0pad, wg0pad, bbpad,
        wcat1, wgcat1, b1[None, :], bg1[None, :],
        wcat2, wgcat2, b2[None, :], bg2[None, :],
        pidx, w3f, wg3f, b3[None, :], bg3[None, :], Wf, bf[None, :])


# speedup vs baseline: 1.0707x; 1.0707x over previous
import functools

import jax
import jax.numpy as jnp
from jax import lax
from jax.experimental import pallas as pl
from jax.experimental.pallas import tpu as pltpu
from jax.experimental.pallas import tpu_sc as plsc

B, N, K, R = 4, 1024, 32, 2
DIMS = [1, 25, 64, 38, 64]
NUM_CLASSES = 10
BN = B * N
NE = BN * K
NW = 32
PPT = BN // NW
EPT = PPT * K
CHK = 128
NCHK = EPT // CHK
PPC = CHK // K
P2T = B * 4 * K
P2PT = P2T // NW

_MESH = plsc.VectorSubcoreMesh(core_axis_name="c", subcore_axis_name="s")
f32 = jnp.float32
i32 = jnp.int32


def _wid():
    return lax.axis_index("s") * 2 + lax.axis_index("c")


def _rsqrt_nr(x):
    ibits = plsc.bitcast(x, i32)
    seed = plsc.bitcast(jnp.int32(0x5F3759DF) - (ibits >> 1), f32)
    y = seed
    for _ in range(3):
        y = y * (1.5 - 0.5 * x * y * y)
    return y


def _sca_body(xc_ref, yc_ref, zc_ref, x0_ref, gidx_ref, rm_ref,
              w0_ref, wg0_ref, bb_ref,
              b0_ref, b1_ref, z1_ref,
              xv, yv, zv, x0v, giv, rmv, b0v, b1v, z1v, wv,
              sem0, sem1, sem2, sem3, sem4, sem5, sem6, sem7):
    wid = _wid()
    base_pt = wid * PPT
    base_e = wid * EPT
    hs = [
        pltpu.async_copy(xc_ref, xv, sem0),
        pltpu.async_copy(yc_ref, yv, sem1),
        pltpu.async_copy(zc_ref, zv, sem2),
        pltpu.async_copy(x0_ref, x0v, sem3),
        pltpu.async_copy(gidx_ref.at[pl.ds(base_e, EPT)], giv, sem4),
        pltpu.async_copy(rm_ref.at[pl.ds(base_e, EPT)], rmv, sem5),
        pltpu.async_copy(w0_ref, wv.at[pl.ds(0, 2)], sem6),
        pltpu.async_copy(wg0_ref, wv.at[pl.ds(2, 2)], sem7),
    ]
    for h in hs:
        h.wait()
    pltpu.sync_copy(bb_ref, wv.at[pl.ds(4, 2)])

    w00a = wv[0, 0:16]
    w00b = wv[0, 16:32]
    w01a = wv[1, 0:16]
    w01b = wv[1, 16:32]
    wg00a = wv[2, 0:16]
    wg00b = wv[2, 16:32]
    wg01a = wv[3, 0:16]
    wg01b = wv[3, 16:32]
    ba = wv[4, 0:16]
    bb = wv[4, 16:32]
    bga = wv[5, 0:16]
    bgb = wv[5, 16:32]
    inv_k = 1.0 / K

    def pt_body(p, _):
        g = base_pt + p
        gs = jnp.full((16,), g, dtype=i32)
        cnx = plsc.load_gather(xv, [gs])
        cny = plsc.load_gather(yv, [gs])
        cnz = plsc.load_gather(zv, [gs])
        a0 = jnp.zeros((16,), f32)
        a1 = jnp.zeros((16,), f32)
        for h in range(2):
            off = p * K + h * 16
            idx = giv[pl.ds(off, 16)]
            jx = plsc.load_gather(xv, [idx])
            jy = plsc.load_gather(yv, [idx])
            jz = plsc.load_gather(zv, [idx])
            xj = plsc.load_gather(x0v, [idx])
            dx = jx - cnx
            dy = jy - cny
            dz = jz - cnz
            d2 = dx * dx + dy * dy + dz * dz + 1e-12
            rm = rmv[pl.ds(off, 16)]
            b0 = jnp.exp(-d2) * rm
            dm1 = d2 * _rsqrt_nr(d2) - 1.0
            b1 = jnp.exp(-(dm1 * dm1)) * rm
            b0v[pl.ds(off, 16)] = b0
            b1v[pl.ds(off, 16)] = b1
            a0 = a0 + b0 * xj
            a1 = a1 + b1 * xj
        s0 = jnp.sum(a0) * inv_k
        s1 = jnp.sum(a1) * inv_k
        msg_a = s0 * w00a + s1 * w01a + ba
        msg_b = s0 * w00b + s1 * w01b + bb
        gmsg_a = s0 * wg00a + s1 * wg01a + bga
        gmsg_b = s0 * wg00b + s1 * wg01b + bgb
        za = jnp.maximum(msg_a, 0.0) / (1.0 + jnp.exp(-gmsg_a))
        zb = jnp.maximum(msg_b, 0.0) / (1.0 + jnp.exp(-gmsg_b))
        z1v[p, 0:16] = za
        z1v[p, 16:32] = zb
        return 0

    lax.fori_loop(0, PPT, pt_body, 0)
    pltpu.sync_copy(b0v, b0_ref.at[pl.ds(base_e, EPT)])
    pltpu.sync_copy(b1v, b1_ref.at[pl.ds(base_e, EPT)])
    pltpu.sync_copy(z1v, z1_ref.at[pl.ds(base_pt, PPT)])


_sca = functools.partial(
    pl.kernel,
    mesh=_MESH,
    compiler_params=pltpu.CompilerParams(needs_layout_passes=False, use_tc_tiling_on_sc=False),
    out_type=(jax.ShapeDtypeStruct((NE,), f32),
              jax.ShapeDtypeStruct((NE,), f32),
              jax.ShapeDtypeStruct((BN, 32), f32)),
    scratch_types=[
        pltpu.VMEM((BN,), f32), pltpu.VMEM((BN,), f32),
        pltpu.VMEM((BN,), f32), pltpu.VMEM((BN,), f32),
        pltpu.VMEM((EPT,), i32), pltpu.VMEM((EPT,), f32),
        pltpu.VMEM((EPT,), f32), pltpu.VMEM((EPT,), f32),
        pltpu.VMEM((PPT, 32), f32),
        pltpu.VMEM((6, 32), f32),
        pltpu.SemaphoreType.DMA, pltpu.SemaphoreType.DMA,
        pltpu.SemaphoreType.DMA, pltpu.SemaphoreType.DMA,
        pltpu.SemaphoreType.DMA, pltpu.SemaphoreType.DMA,
        pltpu.SemaphoreType.DMA, pltpu.SemaphoreType.DMA,
    ],
)(_sca_body)


def _sc2_body(z1_ref, gidx2_ref, b0_ref, b1_ref,
              s1_ref,
              gi2v, b0v, b1v, rows0, rows1, s1v,
              sem0, sem1, semb0, semb1, semb2):
    wid = _wid()
    hg = pltpu.async_copy(gidx2_ref.at[pl.ds(wid * NCHK, NCHK)], gi2v, semb0)
    hb0 = pltpu.async_copy(b0_ref.at[pl.ds(wid * EPT, EPT)], b0v, semb1)
    hb1 = pltpu.async_copy(b1_ref.at[pl.ds(wid * EPT, EPT)], b1v, semb2)
    hg.wait()

    rows = (rows0, rows1)
    sems = (sem0, sem1)
    handles = [None, None]
    handles[0] = pltpu.async_copy(z1_ref.at[gi2v.at[0]], rows0, sem0)
    hb0.wait()
    hb1.wait()
    for c in range(NCHK):
        if c + 1 < NCHK:
            handles[(c + 1) % 2] = pltpu.async_copy(
                z1_ref.at[gi2v.at[c + 1]], rows[(c + 1) % 2],
                sems[(c + 1) % 2])
        handles[c % 2].wait()
        rv = rows[c % 2]

        def pt_body(pp, _):
            p = c * PPC + pp
            zero = jnp.zeros((16,), f32)

            def h_body(h, carry):
                a00, a01, a10, a11 = carry
                b0h = b0v[pl.ds(p * K + h * 16, 16)]
                b1h = b1v[pl.ds(p * K + h * 16, 16)]
                rbase = pp * K + h * 16
                for u in range(16):
                    lo = rv[rbase + u, 0:16]
                    hi = rv[rbase + u, 16:32]
                    b0s = b0h[u]
                    b1s = b1h[u]
                    a00 = a00 + lo * b0s
                    a01 = a01 + hi * b0s
                    a10 = a10 + lo * b1s
                    a11 = a11 + hi * b1s
                return (a00, a01, a10, a11)

            a00, a01, a10, a11 = lax.fori_loop(
                0, 2, h_body, (zero, zero, zero, zero))
            s1v[p, 0:16] = a00
            s1v[p, 16:32] = a01
            s1v[p, 32:48] = a10
            s1v[p, 48:64] = a11
            return 0

        lax.fori_loop(0, PPC, pt_body, 0)
    pltpu.sync_copy(s1v, s1_ref.at[pl.ds(wid * PPT, PPT)])


_sc2 = functools.partial(
    pl.kernel,
    mesh=_MESH,
    compiler_params=pltpu.CompilerParams(needs_layout_passes=False, use_tc_tiling_on_sc=False),
    out_type=jax.ShapeDtypeStruct((BN, 2 * 32), f32),
    scratch_types=[
        pltpu.VMEM((NCHK, CHK), i32),
        pltpu.VMEM((EPT,), f32), pltpu.VMEM((EPT,), f32),
        pltpu.VMEM((CHK, 32), f32), pltpu.VMEM((CHK, 32), f32),
        pltpu.VMEM((PPT, 64), f32),
        pltpu.SemaphoreType.DMA, pltpu.SemaphoreType.DMA,
        pltpu.SemaphoreType.DMA, pltpu.SemaphoreType.DMA,
        pltpu.SemaphoreType.DMA,
    ],
)(_sc2_body)


def _sc3_body(z2_ref, gidxk_ref, b02_ref, b12_ref, pidx_ref,
              s2_ref,
              pidv, girows, b0r, b1r, zr0, zr1, s2v,
              sema, semb, semc, sem0, sem1):
    wid = _wid()
    pltpu.sync_copy(pidx_ref.at[pl.ds(wid * P2PT, P2PT)], pidv)
    hgi = pltpu.async_copy(gidxk_ref.at[pidv], girows, sema)
    hb0 = pltpu.async_copy(b02_ref.at[pidv], b0r, semb)
    hb1 = pltpu.async_copy(b12_ref.at[pidv], b1r, semc)
    hgi.wait()

    zrs = (zr0, zr1)
    sems = (sem0, sem1)
    handles = [None, None]
    handles[0] = pltpu.async_copy(z2_ref.at[girows.at[0]], zr0, sem0)
    hb0.wait()
    hb1.wait()
    for q in range(P2PT):
        if q + 1 < P2PT:
            handles[(q + 1) % 2] = pltpu.async_copy(
                z2_ref.at[girows.at[q + 1]], zrs[(q + 1) % 2],
                sems[(q + 1) % 2])
        handles[q % 2].wait()
        rv = zrs[q % 2]
        zero = jnp.zeros((16,), f32)

        def e_body(e2, carry):
            accs = list(carry)
            for u in range(2):
                e = e2 * 2 + u
                qs = jnp.full((16,), q, dtype=i32)
                es = jnp.full((16,), e, dtype=i32)
                b0s = plsc.load_gather(b0r, [qs, es])
                b1s = plsc.load_gather(b1r, [qs, es])
                for seg in range(4):
                    v = rv[e, pl.ds(seg * 16, 16)]
                    accs[seg] = accs[seg] + v * b0s
                    accs[4 + seg] = accs[4 + seg] + v * b1s
            return tuple(accs)

        accs = lax.fori_loop(0, K // 2, e_body, (zero,) * 8)
        for seg in range(8):
            s2v[q, pl.ds(seg * 16, 16)] = accs[seg]
    pltpu.sync_copy(s2v, s2_ref.at[pl.ds(wid * P2PT, P2PT)])


_sc3 = functools.partial(
    pl.kernel,
    mesh=_MESH,
    compiler_params=pltpu.CompilerParams(needs_layout_passes=False, use_tc_tiling_on_sc=False),
    out_type=jax.ShapeDtypeStruct((P2T, 2 * 64), f32),
    scratch_types=[
        pltpu.VMEM((P2PT,), i32),
        pltpu.VMEM((P2PT, K), i32),
        pltpu.VMEM((P2PT, K), f32), pltpu.VMEM((P2PT, K), f32),
        pltpu.VMEM((K, 64), f32), pltpu.VMEM((K, 64), f32),
        pltpu.VMEM((P2PT, 2 * 64), f32),
        pltpu.SemaphoreType.DMA, pltpu.SemaphoreType.DMA,
        pltpu.SemaphoreType.DMA,
        pltpu.SemaphoreType.DMA, pltpu.SemaphoreType.DMA,
    ],
)(_sc3_body)


def _tc2_body(s1_ref, w_ref, wg_ref, b_ref, bg_ref, z2_ref):
    s1 = s1_ref[...] * (1.0 / K)
    msg = jnp.dot(s1, w_ref[...], preferred_element_type=f32) + b_ref[...]
    gmsg = jnp.dot(s1, wg_ref[...], preferred_element_type=f32) + bg_ref[...]
    z2_ref[...] = jax.nn.relu(msg) * jax.nn.sigmoid(gmsg)


def _tc3_body(s2_ref, w2_ref, wg2_ref, b2_ref, bg2_ref,
              b0t_ref, b1t_ref, w3_ref, wg3_ref, b3_ref, bg3_ref,
              wf_ref, bf_ref, out_ref):
    inv_k = 1.0 / K
    s2 = s2_ref[...] * inv_k
    msg = jnp.dot(s2, w2_ref[...], preferred_element_type=f32) + b2_ref[...]
    gmsg = jnp.dot(s2, wg2_ref[...], preferred_element_type=f32) + bg2_ref[...]
    z3 = jax.nn.relu(msg) * jax.nn.sigmoid(gmsg)

    rows0 = []
    rows1 = []
    for q in range(16):
        blk = z3[q * K:(q + 1) * K, :]
        w0c = b0t_ref[:, q:q + 1]
        w1c = b1t_ref[:, q:q + 1]
        rows0.append(jnp.sum(blk * w0c, axis=0, keepdims=True))
        rows1.append(jnp.sum(blk * w1c, axis=0, keepdims=True))
    s3 = jnp.concatenate(
        [jnp.concatenate(rows0, axis=0),
         jnp.concatenate(rows1, axis=0)], axis=1) * inv_k
    msg3 = jnp.dot(s3, w3_ref[...], preferred_element_type=f32) + b3_ref[...]
    gmsg3 = jnp.dot(s3, wg3_ref[...], preferred_element_type=f32) + bg3_ref[...]
    out4 = jax.nn.relu(msg3) * jax.nn.sigmoid(gmsg3)

    ri = lax.broadcasted_iota(i32, (4, 16), 0)
    ci = lax.broadcasted_iota(i32, (4, 16), 1)
    pmat = jnp.where(ci // 4 == ri, 0.25, 0.0).astype(f32)
    pooled = jnp.dot(pmat, out4, preferred_element_type=f32)
    out_ref[...] = (jnp.dot(pooled, wf_ref[...], preferred_element_type=f32)
                    + bf_ref[...])


@jax.jit
def _forward_impl(xc, yc, zc, x0, gidx, gidx2, gidxk, rmflat,
                  w0pad, wg0pad, bbpad,
                  wcat1, wgcat1, b1r, bg1r,
                  wcat2, wgcat2, b2r, bg2r,
                  pidx, w3f, wg3f, b3r, bg3r, wf, bfr):
    bas0, bas1, z1 = _sca(xc, yc, zc, x0, gidx, rmflat,
                          w0pad, wg0pad, bbpad)

    s1 = _sc2(z1, gidx2, bas0, bas1)

    z2 = pl.pallas_call(
        _tc2_body,
        out_shape=jax.ShapeDtypeStruct((BN, 64), f32),
    )(s1, wcat1, wgcat1, b1r, bg1r)

    s2 = _sc3(z2, gidxk, bas0.reshape(BN, K), bas1.reshape(BN, K), pidx)

    b0t = bas0.reshape(B, N, K)[:, :4, :].reshape(16, K).T
    b1t = bas1.reshape(B, N, K)[:, :4, :].reshape(16, K).T

    out = pl.pallas_call(
        _tc3_body,
        out_shape=jax.ShapeDtypeStruct((B, NUM_CLASSES), f32),
    )(s2, wcat2, wgcat2, b2r, bg2r, b0t, b1t,
      w3f, wg3f, b3r, bg3r, wf, bfr)
    return out


def kernel(input, coords, neighbor, relative_mask,
           W0, Wg0, b0, bg0, W1, Wg1, b1, bg1,
           W2, Wg2, b2, bg2, W3, Wg3, b3, bg3, Wf, bf):
    xc = coords[..., 0].reshape(BN)
    yc = coords[..., 1].reshape(BN)
    zc = coords[..., 2].reshape(BN)
    x0 = input[:, 0, :].reshape(BN)
    nbr = neighbor.astype(i32)
    gidx = (nbr + (jnp.arange(B, dtype=i32) * N)[:, None, None]).reshape(NE)
    gidx2 = gidx.reshape(NE // CHK, CHK)
    gidxk = gidx.reshape(BN, K)
    pidx = gidx.reshape(B, N, K)[:, :4, :].reshape(P2T)
    rmflat = relative_mask.reshape(NE)

    pad7 = jnp.zeros((R, 32 - DIMS[1]), dtype=f32)
    w0pad = jnp.concatenate([W0[:, 0, :], pad7], axis=1)
    wg0pad = jnp.concatenate([Wg0[:, 0, :], pad7], axis=1)
    bbpad = jnp.concatenate(
        [jnp.stack([b0, bg0]), pad7], axis=1)

    z64 = jnp.zeros((64, 64), dtype=f32)
    wcat1 = z64.at[0:25, :].set(W1[0]).at[32:57, :].set(W1[1])
    wgcat1 = z64.at[0:25, :].set(Wg1[0]).at[32:57, :].set(Wg1[1])
    wcat2 = jnp.concatenate([W2[0], W2[1]], axis=0)
    wgcat2 = jnp.concatenate([Wg2[0], Wg2[1]], axis=0)
    w3f = jnp.concatenate([W3[0], W3[1]], axis=0)
    wg3f = jnp.concatenate([Wg3[0], Wg3[1]], axis=0)

    return _forward_impl(
        xc, yc, zc, x0, gidx, gidx2, gidxk, rmflat,
        w0pad, wg0pad, bbpad,
        wcat1, wgcat1, b1[None, :], bg1[None, :],
        wcat2, wgcat2, b2[None, :], bg2[None, :],
        pidx, w3f, wg3f, b3[None, :], bg3[None, :], Wf, bf[None, :])


# speedup vs baseline: 1.1524x; 1.0763x over previous
import functools

import jax
import jax.numpy as jnp
from jax import lax
from jax.experimental import pallas as pl
from jax.experimental.pallas import tpu as pltpu
from jax.experimental.pallas import tpu_sc as plsc

B, N, K, R = 4, 1024, 32, 2
DIMS = [1, 25, 64, 38, 64]
NUM_CLASSES = 10
BN = B * N
NE = BN * K
NW = 32
PPT = BN // NW
EPT = PPT * K
CHK = 128
NCHK = EPT // CHK
PPC = CHK // K
P2T = B * 4 * K
P2PT = P2T // NW

_MESH = plsc.VectorSubcoreMesh(core_axis_name="c", subcore_axis_name="s")
f32 = jnp.float32
i32 = jnp.int32


def _wid():
    return lax.axis_index("s") * 2 + lax.axis_index("c")


def _rsqrt_nr(x):
    ibits = plsc.bitcast(x, i32)
    seed = plsc.bitcast(jnp.int32(0x5F3759DF) - (ibits >> 1), f32)
    y = seed
    for _ in range(3):
        y = y * (1.5 - 0.5 * x * y * y)
    return y


def _sca_body(xc_ref, yc_ref, zc_ref, x0_ref, gidx_ref, rm_ref,
              w0_ref, wg0_ref, bb_ref,
              b0_ref, b1_ref, z1_ref,
              xv, yv, zv, x0v, giv, rmv, b0v, b1v, z1v, wv,
              sem0, sem1, sem2, sem3, sem4, sem5, sem6, sem7):
    wid = _wid()
    base_pt = wid * PPT
    base_e = wid * EPT
    hs = [
        pltpu.async_copy(xc_ref, xv, sem0),
        pltpu.async_copy(yc_ref, yv, sem1),
        pltpu.async_copy(zc_ref, zv, sem2),
        pltpu.async_copy(x0_ref, x0v, sem3),
        pltpu.async_copy(gidx_ref.at[pl.ds(base_e, EPT)], giv, sem4),
        pltpu.async_copy(rm_ref.at[pl.ds(base_e, EPT)], rmv, sem5),
        pltpu.async_copy(w0_ref, wv.at[pl.ds(0, 2)], sem6),
        pltpu.async_copy(wg0_ref, wv.at[pl.ds(2, 2)], sem7),
    ]
    for h in hs:
        h.wait()
    pltpu.sync_copy(bb_ref, wv.at[pl.ds(4, 2)])

    w00a = wv[0, 0:16]
    w00b = wv[0, 16:32]
    w01a = wv[1, 0:16]
    w01b = wv[1, 16:32]
    wg00a = wv[2, 0:16]
    wg00b = wv[2, 16:32]
    wg01a = wv[3, 0:16]
    wg01b = wv[3, 16:32]
    ba = wv[4, 0:16]
    bb = wv[4, 16:32]
    bga = wv[5, 0:16]
    bgb = wv[5, 16:32]
    inv_k = 1.0 / K

    def pt_body(p, _):
        g = base_pt + p
        gs = jnp.full((16,), g, dtype=i32)
        cnx = plsc.load_gather(xv, [gs])
        cny = plsc.load_gather(yv, [gs])
        cnz = plsc.load_gather(zv, [gs])
        a0 = jnp.zeros((16,), f32)
        a1 = jnp.zeros((16,), f32)
        for h in range(2):
            off = p * K + h * 16
            idx = giv[pl.ds(off, 16)]
            jx = plsc.load_gather(xv, [idx])
            jy = plsc.load_gather(yv, [idx])
            jz = plsc.load_gather(zv, [idx])
            xj = plsc.load_gather(x0v, [idx])
            dx = jx - cnx
            dy = jy - cny
            dz = jz - cnz
            d2 = dx * dx + dy * dy + dz * dz + 1e-12
            rm = rmv[pl.ds(off, 16)]
            b0 = jnp.exp(-d2) * rm
            dm1 = d2 * _rsqrt_nr(d2) - 1.0
            b1 = jnp.exp(-(dm1 * dm1)) * rm
            b0v[pl.ds(off, 16)] = b0
            b1v[pl.ds(off, 16)] = b1
            a0 = a0 + b0 * xj
            a1 = a1 + b1 * xj
        s0 = jnp.sum(a0) * inv_k
        s1 = jnp.sum(a1) * inv_k
        msg_a = s0 * w00a + s1 * w01a + ba
        msg_b = s0 * w00b + s1 * w01b + bb
        gmsg_a = s0 * wg00a + s1 * wg01a + bga
        gmsg_b = s0 * wg00b + s1 * wg01b + bgb
        za = jnp.maximum(msg_a, 0.0) / (1.0 + jnp.exp(-gmsg_a))
        zb = jnp.maximum(msg_b, 0.0) / (1.0 + jnp.exp(-gmsg_b))
        z1v[p, 0:16] = za
        z1v[p, 16:32] = zb
        return 0

    lax.fori_loop(0, PPT, pt_body, 0)
    pltpu.sync_copy(b0v, b0_ref.at[pl.ds(base_e, EPT)])
    pltpu.sync_copy(b1v, b1_ref.at[pl.ds(base_e, EPT)])
    pltpu.sync_copy(z1v, z1_ref.at[pl.ds(base_pt, PPT)])


_sca = functools.partial(
    pl.kernel,
    mesh=_MESH,
    compiler_params=pltpu.CompilerParams(needs_layout_passes=False, use_tc_tiling_on_sc=False),
    out_type=(jax.ShapeDtypeStruct((NE,), f32),
              jax.ShapeDtypeStruct((NE,), f32),
              jax.ShapeDtypeStruct((BN, 32), f32)),
    scratch_types=[
        pltpu.VMEM((BN,), f32), pltpu.VMEM((BN,), f32),
        pltpu.VMEM((BN,), f32), pltpu.VMEM((BN,), f32),
        pltpu.VMEM((EPT,), i32), pltpu.VMEM((EPT,), f32),
        pltpu.VMEM((EPT,), f32), pltpu.VMEM((EPT,), f32),
        pltpu.VMEM((PPT, 32), f32),
        pltpu.VMEM((6, 32), f32),
        pltpu.SemaphoreType.DMA, pltpu.SemaphoreType.DMA,
        pltpu.SemaphoreType.DMA, pltpu.SemaphoreType.DMA,
        pltpu.SemaphoreType.DMA, pltpu.SemaphoreType.DMA,
        pltpu.SemaphoreType.DMA, pltpu.SemaphoreType.DMA,
    ],
)(_sca_body)


def _sc2_body(z1_ref, gidx2_ref, b0_ref, b1_ref,
              s1_ref,
              gi2v, b0v, b1v, rows0, rows1, s1v, z1loc, z1sh,
              sem0, sem1, semb0, semb1, semb2, semz):
    wid = _wid()
    sid = lax.axis_index("s")
    hz = pltpu.async_copy(z1_ref.at[pl.ds(sid * (BN // 16), BN // 16)],
                          z1loc, semz)
    hg = pltpu.async_copy(gidx2_ref.at[pl.ds(wid * NCHK, NCHK)], gi2v, semb0)
    hb0 = pltpu.async_copy(b0_ref.at[pl.ds(wid * EPT, EPT)], b0v, semb1)
    hb1 = pltpu.async_copy(b1_ref.at[pl.ds(wid * EPT, EPT)], b1v, semb2)
    hz.wait()
    pltpu.sync_copy(z1loc, z1sh.at[pl.ds(sid * (BN // 16), BN // 16)])
    plsc.subcore_barrier()
    hg.wait()

    rows = (rows0, rows1)
    sems = (sem0, sem1)
    handles = [None, None]
    handles[0] = pltpu.async_copy(z1sh.at[gi2v.at[0]], rows0, sem0)
    hb0.wait()
    hb1.wait()
    for c in range(NCHK):
        if c + 1 < NCHK:
            handles[(c + 1) % 2] = pltpu.async_copy(
                z1sh.at[gi2v.at[c + 1]], rows[(c + 1) % 2],
                sems[(c + 1) % 2])
        handles[c % 2].wait()
        rv = rows[c % 2]

        def pt_body(pp, _):
            p = c * PPC + pp
            zero = jnp.zeros((16,), f32)

            def h_body(h, carry):
                a00, a01, a10, a11 = carry
                b0h = b0v[pl.ds(p * K + h * 16, 16)]
                b1h = b1v[pl.ds(p * K + h * 16, 16)]
                rbase = pp * K + h * 16
                for u in range(16):
                    lo = rv[rbase + u, 0:16]
                    hi = rv[rbase + u, 16:32]
                    b0s = b0h[u]
                    b1s = b1h[u]
                    a00 = a00 + lo * b0s
                    a01 = a01 + hi * b0s
                    a10 = a10 + lo * b1s
                    a11 = a11 + hi * b1s
                return (a00, a01, a10, a11)

            a00, a01, a10, a11 = lax.fori_loop(
                0, 2, h_body, (zero, zero, zero, zero))
            s1v[p, 0:16] = a00
            s1v[p, 16:32] = a01
            s1v[p, 32:48] = a10
            s1v[p, 48:64] = a11
            return 0

        lax.fori_loop(0, PPC, pt_body, 0)
    pltpu.sync_copy(s1v, s1_ref.at[pl.ds(wid * PPT, PPT)])


_sc2 = functools.partial(
    pl.kernel,
    mesh=_MESH,
    compiler_params=pltpu.CompilerParams(needs_layout_passes=False, use_tc_tiling_on_sc=False),
    out_type=jax.ShapeDtypeStruct((BN, 2 * 32), f32),
    scratch_types=[
        pltpu.VMEM((NCHK, CHK), i32),
        pltpu.VMEM((EPT,), f32), pltpu.VMEM((EPT,), f32),
        pltpu.VMEM((CHK, 32), f32), pltpu.VMEM((CHK, 32), f32),
        pltpu.VMEM((PPT, 64), f32),
        pltpu.VMEM((BN // 16, 32), f32),
        pltpu.VMEM_SHARED((BN, 32), f32),
        pltpu.SemaphoreType.DMA, pltpu.SemaphoreType.DMA,
        pltpu.SemaphoreType.DMA, pltpu.SemaphoreType.DMA,
        pltpu.SemaphoreType.DMA, pltpu.SemaphoreType.DMA,
    ],
)(_sc2_body)


def _sc3_body(z2_ref, gidxk_ref, b02_ref, b12_ref, pidx_ref,
              s2_ref,
              pidv, girows, b0r, b1r, zr0, zr1, s2v,
              sema, semb, semc, sem0, sem1):
    wid = _wid()
    pltpu.sync_copy(pidx_ref.at[pl.ds(wid * P2PT, P2PT)], pidv)
    hgi = pltpu.async_copy(gidxk_ref.at[pidv], girows, sema)
    hb0 = pltpu.async_copy(b02_ref.at[pidv], b0r, semb)
    hb1 = pltpu.async_copy(b12_ref.at[pidv], b1r, semc)
    hgi.wait()

    zrs = (zr0, zr1)
    sems = (sem0, sem1)
    handles = [None, None]
    handles[0] = pltpu.async_copy(z2_ref.at[girows.at[0]], zr0, sem0)
    hb0.wait()
    hb1.wait()
    for q in range(P2PT):
        if q + 1 < P2PT:
            handles[(q + 1) % 2] = pltpu.async_copy(
                z2_ref.at[girows.at[q + 1]], zrs[(q + 1) % 2],
                sems[(q + 1) % 2])
        handles[q % 2].wait()
        rv = zrs[q % 2]
        zero = jnp.zeros((16,), f32)

        def e_body(e2, carry):
            accs = list(carry)
            for u in range(2):
                e = e2 * 2 + u
                qs = jnp.full((16,), q, dtype=i32)
                es = jnp.full((16,), e, dtype=i32)
                b0s = plsc.load_gather(b0r, [qs, es])
                b1s = plsc.load_gather(b1r, [qs, es])
                for seg in range(4):
                    v = rv[e, pl.ds(seg * 16, 16)]
                    accs[seg] = accs[seg] + v * b0s
                    accs[4 + seg] = accs[4 + seg] + v * b1s
            return tuple(accs)

        accs = lax.fori_loop(0, K // 2, e_body, (zero,) * 8)
        for seg in range(8):
            s2v[q, pl.ds(seg * 16, 16)] = accs[seg]
    pltpu.sync_copy(s2v, s2_ref.at[pl.ds(wid * P2PT, P2PT)])


_sc3 = functools.partial(
    pl.kernel,
    mesh=_MESH,
    compiler_params=pltpu.CompilerParams(needs_layout_passes=False, use_tc_tiling_on_sc=False),
    out_type=jax.ShapeDtypeStruct((P2T, 2 * 64), f32),
    scratch_types=[
        pltpu.VMEM((P2PT,), i32),
        pltpu.VMEM((P2PT, K), i32),
        pltpu.VMEM((P2PT, K), f32), pltpu.VMEM((P2PT, K), f32),
        pltpu.VMEM((K, 64), f32), pltpu.VMEM((K, 64), f32),
        pltpu.VMEM((P2PT, 2 * 64), f32),
        pltpu.SemaphoreType.DMA, pltpu.SemaphoreType.DMA,
        pltpu.SemaphoreType.DMA,
        pltpu.SemaphoreType.DMA, pltpu.SemaphoreType.DMA,
    ],
)(_sc3_body)


def _tc2_body(s1_ref, w_ref, wg_ref, b_ref, bg_ref, z2_ref):
    s1 = s1_ref[...] * (1.0 / K)
    msg = jnp.dot(s1, w_ref[...], preferred_element_type=f32) + b_ref[...]
    gmsg = jnp.dot(s1, wg_ref[...], preferred_element_type=f32) + bg_ref[...]
    z2_ref[...] = jax.nn.relu(msg) * jax.nn.sigmoid(gmsg)


def _tc3_body(s2_ref, w2_ref, wg2_ref, b2_ref, bg2_ref,
              b0t_ref, b1t_ref, w3_ref, wg3_ref, b3_ref, bg3_ref,
              wf_ref, bf_ref, out_ref):
    inv_k = 1.0 / K
    s2 = s2_ref[...] * inv_k
    msg = jnp.dot(s2, w2_ref[...], preferred_element_type=f32) + b2_ref[...]
    gmsg = jnp.dot(s2, wg2_ref[...], preferred_element_type=f32) + bg2_ref[...]
    z3 = jax.nn.relu(msg) * jax.nn.sigmoid(gmsg)

    rows0 = []
    rows1 = []
    for q in range(16):
        blk = z3[q * K:(q + 1) * K, :]
        w0c = b0t_ref[:, q:q + 1]
        w1c = b1t_ref[:, q:q + 1]
        rows0.append(jnp.sum(blk * w0c, axis=0, keepdims=True))
        rows1.append(jnp.sum(blk * w1c, axis=0, keepdims=True))
    s3 = jnp.concatenate(
        [jnp.concatenate(rows0, axis=0),
         jnp.concatenate(rows1, axis=0)], axis=1) * inv_k
    msg3 = jnp.dot(s3, w3_ref[...], preferred_element_type=f32) + b3_ref[...]
    gmsg3 = jnp.dot(s3, wg3_ref[...], preferred_element_type=f32) + bg3_ref[...]
    out4 = jax.nn.relu(msg3) * jax.nn.sigmoid(gmsg3)

    ri = lax.broadcasted_iota(i32, (4, 16), 0)
    ci = lax.broadcasted_iota(i32, (4, 16), 1)
    pmat = jnp.where(ci // 4 == ri, 0.25, 0.0).astype(f32)
    pooled = jnp.dot(pmat, out4, preferred_element_type=f32)
    out_ref[...] = (jnp.dot(pooled, wf_ref[...], preferred_element_type=f32)
                    + bf_ref[...])


@jax.jit
def _forward_impl(xc, yc, zc, x0, gidx, gidx2, gidxk, rmflat,
                  w0pad, wg0pad, bbpad,
                  wcat1, wgcat1, b1r, bg1r,
                  wcat2, wgcat2, b2r, bg2r,
                  pidx, w3f, wg3f, b3r, bg3r, wf, bfr):
    bas0, bas1, z1 = _sca(xc, yc, zc, x0, gidx, rmflat,
                          w0pad, wg0pad, bbpad)

    s1 = _sc2(z1, gidx2, bas0, bas1)

    z2 = pl.pallas_call(
        _tc2_body,
        out_shape=jax.ShapeDtypeStruct((BN, 64), f32),
    )(s1, wcat1, wgcat1, b1r, bg1r)

    s2 = _sc3(z2, gidxk, bas0.reshape(BN, K), bas1.reshape(BN, K), pidx)

    b0t = bas0.reshape(B, N, K)[:, :4, :].reshape(16, K).T
    b1t = bas1.reshape(B, N, K)[:, :4, :].reshape(16, K).T

    out = pl.pallas_call(
        _tc3_body,
        out_shape=jax.ShapeDtypeStruct((B, NUM_CLASSES), f32),
    )(s2, wcat2, wgcat2, b2r, bg2r, b0t, b1t,
      w3f, wg3f, b3r, bg3r, wf, bfr)
    return out


def kernel(input, coords, neighbor, relative_mask,
           W0, Wg0, b0, bg0, W1, Wg1, b1, bg1,
           W2, Wg2, b2, bg2, W3, Wg3, b3, bg3, Wf, bf):
    xc = coords[..., 0].reshape(BN)
    yc = coords[..., 1].reshape(BN)
    zc = coords[..., 2].reshape(BN)
    x0 = input[:, 0, :].reshape(BN)
    nbr = neighbor.astype(i32)
    gidx = (nbr + (jnp.arange(B, dtype=i32) * N)[:, None, None]).reshape(NE)
    gidx2 = gidx.reshape(NE // CHK, CHK)
    gidxk = gidx.reshape(BN, K)
    pidx = gidx.reshape(B, N, K)[:, :4, :].reshape(P2T)
    rmflat = relative_mask.reshape(NE)

    pad7 = jnp.zeros((R, 32 - DIMS[1]), dtype=f32)
    w0pad = jnp.concatenate([W0[:, 0, :], pad7], axis=1)
    wg0pad = jnp.concatenate([Wg0[:, 0, :], pad7], axis=1)
    bbpad = jnp.concatenate(
        [jnp.stack([b0, bg0]), pad7], axis=1)

    z64 = jnp.zeros((64, 64), dtype=f32)
    wcat1 = z64.at[0:25, :].set(W1[0]).at[32:57, :].set(W1[1])
    wgcat1 = z64.at[0:25, :].set(Wg1[0]).at[32:57, :].set(Wg1[1])
    wcat2 = jnp.concatenate([W2[0], W2[1]], axis=0)
    wgcat2 = jnp.concatenate([Wg2[0], Wg2[1]], axis=0)
    w3f = jnp.concatenate([W3[0], W3[1]], axis=0)
    wg3f = jnp.concatenate([Wg3[0], Wg3[1]], axis=0)

    return _forward_impl(
        xc, yc, zc, x0, gidx, gidx2, gidxk, rmflat,
        w0pad, wg0pad, bbpad,
        wcat1, wgcat1, b1[None, :], bg1[None, :],
        wcat2, wgcat2, b2[None, :], bg2[None, :],
        pidx, w3f, wg3f, b3[None, :], bg3[None, :], Wf, bf[None, :])


# speedup vs baseline: 1.1881x; 1.0310x over previous
import functools

import jax
import jax.numpy as jnp
from jax import lax
from jax.experimental import pallas as pl
from jax.experimental.pallas import tpu as pltpu
from jax.experimental.pallas import tpu_sc as plsc

B, N, K, R = 4, 1024, 32, 2
DIMS = [1, 25, 64, 38, 64]
NUM_CLASSES = 10
BN = B * N
NE = BN * K
NW = 32
PPT = BN // NW
EPT = PPT * K
CHK = 128
NCHK = EPT // CHK
PPC = CHK // K
P2T = B * 4 * K
P2PT = P2T // NW

_MESH = plsc.VectorSubcoreMesh(core_axis_name="c", subcore_axis_name="s")
f32 = jnp.float32
i32 = jnp.int32


def _wid():
    return lax.axis_index("s") * 2 + lax.axis_index("c")


def _rsqrt_nr(x):
    ibits = plsc.bitcast(x, i32)
    seed = plsc.bitcast(jnp.int32(0x5F3759DF) - (ibits >> 1), f32)
    y = seed
    for _ in range(3):
        y = y * (1.5 - 0.5 * x * y * y)
    return y


def _sca_body(xc_ref, yc_ref, zc_ref, x0_ref, gidx_ref, rm_ref,
              w0_ref, wg0_ref, bb_ref,
              b0_ref, b1_ref, z1_ref,
              xv, yv, zv, x0v, giv, rmv, b0v, b1v, z1v, wv,
              sem0, sem1, sem2, sem3, sem4, sem5, sem6, sem7):
    wid = _wid()
    base_pt = wid * PPT
    base_e = wid * EPT
    hs = [
        pltpu.async_copy(xc_ref, xv, sem0),
        pltpu.async_copy(yc_ref, yv, sem1),
        pltpu.async_copy(zc_ref, zv, sem2),
        pltpu.async_copy(x0_ref, x0v, sem3),
        pltpu.async_copy(gidx_ref.at[pl.ds(base_e, EPT)], giv, sem4),
        pltpu.async_copy(rm_ref.at[pl.ds(base_e, EPT)], rmv, sem5),
        pltpu.async_copy(w0_ref, wv.at[pl.ds(0, 2)], sem6),
        pltpu.async_copy(wg0_ref, wv.at[pl.ds(2, 2)], sem7),
    ]
    for h in hs:
        h.wait()
    pltpu.sync_copy(bb_ref, wv.at[pl.ds(4, 2)])

    w00a = wv[0, 0:16]
    w00b = wv[0, 16:32]
    w01a = wv[1, 0:16]
    w01b = wv[1, 16:32]
    wg00a = wv[2, 0:16]
    wg00b = wv[2, 16:32]
    wg01a = wv[3, 0:16]
    wg01b = wv[3, 16:32]
    ba = wv[4, 0:16]
    bb = wv[4, 16:32]
    bga = wv[5, 0:16]
    bgb = wv[5, 16:32]
    inv_k = 1.0 / K

    def pt_body(p, _):
        g = base_pt + p
        gs = jnp.full((16,), g, dtype=i32)
        cnx = plsc.load_gather(xv, [gs])
        cny = plsc.load_gather(yv, [gs])
        cnz = plsc.load_gather(zv, [gs])
        a0 = jnp.zeros((16,), f32)
        a1 = jnp.zeros((16,), f32)
        for h in range(2):
            off = p * K + h * 16
            idx = giv[pl.ds(off, 16)]
            jx = plsc.load_gather(xv, [idx])
            jy = plsc.load_gather(yv, [idx])
            jz = plsc.load_gather(zv, [idx])
            xj = plsc.load_gather(x0v, [idx])
            dx = jx - cnx
            dy = jy - cny
            dz = jz - cnz
            d2 = dx * dx + dy * dy + dz * dz + 1e-12
            rm = rmv[pl.ds(off, 16)]
            b0 = jnp.exp(-d2) * rm
            dm1 = d2 * _rsqrt_nr(d2) - 1.0
            b1 = jnp.exp(-(dm1 * dm1)) * rm
            b0v[pl.ds(off, 16)] = b0
            b1v[pl.ds(off, 16)] = b1
            a0 = a0 + b0 * xj
            a1 = a1 + b1 * xj
        s0 = jnp.sum(a0) * inv_k
        s1 = jnp.sum(a1) * inv_k
        msg_a = s0 * w00a + s1 * w01a + ba
        msg_b = s0 * w00b + s1 * w01b + bb
        gmsg_a = s0 * wg00a + s1 * wg01a + bga
        gmsg_b = s0 * wg00b + s1 * wg01b + bgb
        za = jnp.maximum(msg_a, 0.0) / (1.0 + jnp.exp(-gmsg_a))
        zb = jnp.maximum(msg_b, 0.0) / (1.0 + jnp.exp(-gmsg_b))
        z1v[p, 0:16] = za
        z1v[p, 16:32] = zb
        return 0

    lax.fori_loop(0, PPT, pt_body, 0)
    pltpu.sync_copy(b0v, b0_ref.at[pl.ds(base_e, EPT)])
    pltpu.sync_copy(b1v, b1_ref.at[pl.ds(base_e, EPT)])
    pltpu.sync_copy(z1v, z1_ref.at[pl.ds(base_pt, PPT)])


_sca = functools.partial(
    pl.kernel,
    mesh=_MESH,
    compiler_params=pltpu.CompilerParams(needs_layout_passes=False, use_tc_tiling_on_sc=False),
    out_type=(jax.ShapeDtypeStruct((NE,), f32),
              jax.ShapeDtypeStruct((NE,), f32),
              jax.ShapeDtypeStruct((BN, 32), f32)),
    scratch_types=[
        pltpu.VMEM((BN,), f32), pltpu.VMEM((BN,), f32),
        pltpu.VMEM((BN,), f32), pltpu.VMEM((BN,), f32),
        pltpu.VMEM((EPT,), i32), pltpu.VMEM((EPT,), f32),
        pltpu.VMEM((EPT,), f32), pltpu.VMEM((EPT,), f32),
        pltpu.VMEM((PPT, 32), f32),
        pltpu.VMEM((6, 32), f32),
        pltpu.SemaphoreType.DMA, pltpu.SemaphoreType.DMA,
        pltpu.SemaphoreType.DMA, pltpu.SemaphoreType.DMA,
        pltpu.SemaphoreType.DMA, pltpu.SemaphoreType.DMA,
        pltpu.SemaphoreType.DMA, pltpu.SemaphoreType.DMA,
    ],
)(_sca_body)


def _sc2_body(z1_ref, gidx2_ref, b0_ref, b1_ref,
              s1_ref,
              gi2v, b0v, b1v, rows0, rows1, s1v, z1loc, z1sh,
              sem0, sem1, semb0, semb1, semb2, semz):
    wid = _wid()
    sid = lax.axis_index("s")
    hz = pltpu.async_copy(z1_ref.at[pl.ds(sid * (BN // 16), BN // 16)],
                          z1loc, semz)
    hg = pltpu.async_copy(gidx2_ref.at[pl.ds(wid * NCHK, NCHK)], gi2v, semb0)
    hb0 = pltpu.async_copy(b0_ref.at[pl.ds(wid * EPT, EPT)], b0v, semb1)
    hb1 = pltpu.async_copy(b1_ref.at[pl.ds(wid * EPT, EPT)], b1v, semb2)
    hz.wait()
    pltpu.sync_copy(z1loc, z1sh.at[pl.ds(sid * (BN // 16), BN // 16)])
    plsc.subcore_barrier()
    hg.wait()

    rows = (rows0, rows1)
    sems = (sem0, sem1)
    handles = [None, None]
    handles[0] = pltpu.async_copy(z1sh.at[gi2v.at[0]], rows0, sem0)
    hb0.wait()
    hb1.wait()
    for c in range(NCHK):
        if c + 1 < NCHK:
            handles[(c + 1) % 2] = pltpu.async_copy(
                z1sh.at[gi2v.at[c + 1]], rows[(c + 1) % 2],
                sems[(c + 1) % 2])
        handles[c % 2].wait()
        rv = rows[c % 2]

        def pt_body(pp, _):
            p = c * PPC + pp
            zero = jnp.zeros((16,), f32)

            def h_body(h, carry):
                a00, a01, a10, a11 = carry
                b0h = b0v[pl.ds(p * K + h * 16, 16)]
                b1h = b1v[pl.ds(p * K + h * 16, 16)]
                rbase = pp * K + h * 16
                for u in range(16):
                    lo = rv[rbase + u, 0:16]
                    hi = rv[rbase + u, 16:32]
                    b0s = b0h[u]
                    b1s = b1h[u]
                    a00 = a00 + lo * b0s
                    a01 = a01 + hi * b0s
                    a10 = a10 + lo * b1s
                    a11 = a11 + hi * b1s
                return (a00, a01, a10, a11)

            a00, a01, a10, a11 = lax.fori_loop(
                0, 2, h_body, (zero, zero, zero, zero))
            s1v[p, 0:16] = a00
            s1v[p, 16:32] = a01
            s1v[p, 32:48] = a10
            s1v[p, 48:64] = a11
            return 0

        lax.fori_loop(0, PPC, pt_body, 0)
    pltpu.sync_copy(s1v, s1_ref.at[pl.ds(wid * PPT, PPT)])


_sc2 = functools.partial(
    pl.kernel,
    mesh=_MESH,
    compiler_params=pltpu.CompilerParams(needs_layout_passes=False, use_tc_tiling_on_sc=False),
    out_type=jax.ShapeDtypeStruct((BN, 2 * 32), f32),
    scratch_types=[
        pltpu.VMEM((NCHK, CHK), i32),
        pltpu.VMEM((EPT,), f32), pltpu.VMEM((EPT,), f32),
        pltpu.VMEM((CHK, 32), f32), pltpu.VMEM((CHK, 32), f32),
        pltpu.VMEM((PPT, 64), f32),
        pltpu.VMEM((BN // 16, 32), f32),
        pltpu.VMEM_SHARED((BN, 32), f32),
        pltpu.SemaphoreType.DMA, pltpu.SemaphoreType.DMA,
        pltpu.SemaphoreType.DMA, pltpu.SemaphoreType.DMA,
        pltpu.SemaphoreType.DMA, pltpu.SemaphoreType.DMA,
    ],
)(_sc2_body)


def _sc3_body(z2_ref, gidxk_ref, b02_ref, b12_ref, pidx_ref,
              s2_ref,
              pidv, girows, b0r, b1r, zr0, zr1, s2v, z2loc, z2sh,
              sema, semb, semc, sem0, sem1, semz):
    wid = _wid()
    sid = lax.axis_index("s")
    hz = pltpu.async_copy(z2_ref.at[pl.ds(sid * (BN // 16), BN // 16)],
                          z2loc, semz)
    pltpu.sync_copy(pidx_ref.at[pl.ds(wid * P2PT, P2PT)], pidv)
    hgi = pltpu.async_copy(gidxk_ref.at[pidv], girows, sema)
    hb0 = pltpu.async_copy(b02_ref.at[pidv], b0r, semb)
    hb1 = pltpu.async_copy(b12_ref.at[pidv], b1r, semc)
    hz.wait()
    pltpu.sync_copy(z2loc, z2sh.at[pl.ds(sid * (BN // 16), BN // 16)])
    plsc.subcore_barrier()
    hgi.wait()

    zrs = (zr0, zr1)
    sems = (sem0, sem1)
    handles = [None, None]
    handles[0] = pltpu.async_copy(z2sh.at[girows.at[0]], zr0, sem0)
    hb0.wait()
    hb1.wait()
    for q in range(P2PT):
        if q + 1 < P2PT:
            handles[(q + 1) % 2] = pltpu.async_copy(
                z2sh.at[girows.at[q + 1]], zrs[(q + 1) % 2],
                sems[(q + 1) % 2])
        handles[q % 2].wait()
        rv = zrs[q % 2]
        zero = jnp.zeros((16,), f32)

        def e_body(e2, carry):
            accs = list(carry)
            for u in range(2):
                e = e2 * 2 + u
                qs = jnp.full((16,), q, dtype=i32)
                es = jnp.full((16,), e, dtype=i32)
                b0s = plsc.load_gather(b0r, [qs, es])
                b1s = plsc.load_gather(b1r, [qs, es])
                for seg in range(4):
                    v = rv[e, pl.ds(seg * 16, 16)]
                    accs[seg] = accs[seg] + v * b0s
                    accs[4 + seg] = accs[4 + seg] + v * b1s
            return tuple(accs)

        accs = lax.fori_loop(0, K // 2, e_body, (zero,) * 8)
        for seg in range(8):
            s2v[q, pl.ds(seg * 16, 16)] = accs[seg]
    pltpu.sync_copy(s2v, s2_ref.at[pl.ds(wid * P2PT, P2PT)])


_sc3 = functools.partial(
    pl.kernel,
    mesh=_MESH,
    compiler_params=pltpu.CompilerParams(needs_layout_passes=False, use_tc_tiling_on_sc=False),
    out_type=jax.ShapeDtypeStruct((P2T, 2 * 64), f32),
    scratch_types=[
        pltpu.VMEM((P2PT,), i32),
        pltpu.VMEM((P2PT, K), i32),
        pltpu.VMEM((P2PT, K), f32), pltpu.VMEM((P2PT, K), f32),
        pltpu.VMEM((K, 64), f32), pltpu.VMEM((K, 64), f32),
        pltpu.VMEM((P2PT, 2 * 64), f32),
        pltpu.VMEM((BN // 16, 64), f32),
        pltpu.VMEM_SHARED((BN, 64), f32),
        pltpu.SemaphoreType.DMA, pltpu.SemaphoreType.DMA,
        pltpu.SemaphoreType.DMA,
        pltpu.SemaphoreType.DMA, pltpu.SemaphoreType.DMA,
        pltpu.SemaphoreType.DMA,
    ],
)(_sc3_body)


def _tc2_body(s1_ref, w_ref, wg_ref, b_ref, bg_ref, z2_ref):
    s1 = s1_ref[...] * (1.0 / K)
    msg = jnp.dot(s1, w_ref[...], preferred_element_type=f32) + b_ref[...]
    gmsg = jnp.dot(s1, wg_ref[...], preferred_element_type=f32) + bg_ref[...]
    z2_ref[...] = jax.nn.relu(msg) * jax.nn.sigmoid(gmsg)


def _tc3_body(s2_ref, w2_ref, wg2_ref, b2_ref, bg2_ref,
              b0t_ref, b1t_ref, w3_ref, wg3_ref, b3_ref, bg3_ref,
              wf_ref, bf_ref, out_ref):
    inv_k = 1.0 / K
    s2 = s2_ref[...] * inv_k
    msg = jnp.dot(s2, w2_ref[...], preferred_element_type=f32) + b2_ref[...]
    gmsg = jnp.dot(s2, wg2_ref[...], preferred_element_type=f32) + bg2_ref[...]
    z3 = jax.nn.relu(msg) * jax.nn.sigmoid(gmsg)

    rows0 = []
    rows1 = []
    for q in range(16):
        blk = z3[q * K:(q + 1) * K, :]
        w0c = b0t_ref[:, q:q + 1]
        w1c = b1t_ref[:, q:q + 1]
        rows0.append(jnp.sum(blk * w0c, axis=0, keepdims=True))
        rows1.append(jnp.sum(blk * w1c, axis=0, keepdims=True))
    s3 = jnp.concatenate(
        [jnp.concatenate(rows0, axis=0),
         jnp.concatenate(rows1, axis=0)], axis=1) * inv_k
    msg3 = jnp.dot(s3, w3_ref[...], preferred_element_type=f32) + b3_ref[...]
    gmsg3 = jnp.dot(s3, wg3_ref[...], preferred_element_type=f32) + bg3_ref[...]
    out4 = jax.nn.relu(msg3) * jax.nn.sigmoid(gmsg3)

    ri = lax.broadcasted_iota(i32, (4, 16), 0)
    ci = lax.broadcasted_iota(i32, (4, 16), 1)
    pmat = jnp.where(ci // 4 == ri, 0.25, 0.0).astype(f32)
    pooled = jnp.dot(pmat, out4, preferred_element_type=f32)
    out_ref[...] = (jnp.dot(pooled, wf_ref[...], preferred_element_type=f32)
                    + bf_ref[...])


@jax.jit
def _forward_impl(xc, yc, zc, x0, gidx, gidx2, gidxk, rmflat,
                  w0pad, wg0pad, bbpad,
                  wcat1, wgcat1, b1r, bg1r,
                  wcat2, wgcat2, b2r, bg2r,
                  pidx, w3f, wg3f, b3r, bg3r, wf, bfr):
    bas0, bas1, z1 = _sca(xc, yc, zc, x0, gidx, rmflat,
                          w0pad, wg0pad, bbpad)

    s1 = _sc2(z1, gidx2, bas0, bas1)

    z2 = pl.pallas_call(
        _tc2_body,
        out_shape=jax.ShapeDtypeStruct((BN, 64), f32),
    )(s1, wcat1, wgcat1, b1r, bg1r)

    s2 = _sc3(z2, gidxk, bas0.reshape(BN, K), bas1.reshape(BN, K), pidx)

    b0t = bas0.reshape(B, N, K)[:, :4, :].reshape(16, K).T
    b1t = bas1.reshape(B, N, K)[:, :4, :].reshape(16, K).T

    out = pl.pallas_call(
        _tc3_body,
        out_shape=jax.ShapeDtypeStruct((B, NUM_CLASSES), f32),
    )(s2, wcat2, wgcat2, b2r, bg2r, b0t, b1t,
      w3f, wg3f, b3r, bg3r, wf, bfr)
    return out


def kernel(input, coords, neighbor, relative_mask,
           W0, Wg0, b0, bg0, W1, Wg1, b1, bg1,
           W2, Wg2, b2, bg2, W3, Wg3, b3, bg3, Wf, bf):
    xc = coords[..., 0].reshape(BN)
    yc = coords[..., 1].reshape(BN)
    zc = coords[..., 2].reshape(BN)
    x0 = input[:, 0, :].reshape(BN)
    nbr = neighbor.astype(i32)
    gidx = (nbr + (jnp.arange(B, dtype=i32) * N)[:, None, None]).reshape(NE)
    gidx2 = gidx.reshape(NE // CHK, CHK)
    gidxk = gidx.reshape(BN, K)
    pidx = gidx.reshape(B, N, K)[:, :4, :].reshape(P2T)
    rmflat = relative_mask.reshape(NE)

    pad7 = jnp.zeros((R, 32 - DIMS[1]), dtype=f32)
    w0pad = jnp.concatenate([W0[:, 0, :], pad7], axis=1)
    wg0pad = jnp.concatenate([Wg0[:, 0, :], pad7], axis=1)
    bbpad = jnp.concatenate(
        [jnp.stack([b0, bg0]), pad7], axis=1)

    z64 = jnp.zeros((64, 64), dtype=f32)
    wcat1 = z64.at[0:25, :].set(W1[0]).at[32:57, :].set(W1[1])
    wgcat1 = z64.at[0:25, :].set(Wg1[0]).at[32:57, :].set(Wg1[1])
    wcat2 = jnp.concatenate([W2[0], W2[1]], axis=0)
    wgcat2 = jnp.concatenate([Wg2[0], Wg2[1]], axis=0)
    w3f = jnp.concatenate([W3[0], W3[1]], axis=0)
    wg3f = jnp.concatenate([Wg3[0], Wg3[1]], axis=0)

    return _forward_impl(
        xc, yc, zc, x0, gidx, gidx2, gidxk, rmflat,
        w0pad, wg0pad, bbpad,
        wcat1, wgcat1, b1[None, :], bg1[None, :],
        wcat2, wgcat2, b2[None, :], bg2[None, :],
        pidx, w3f, wg3f, b3[None, :], bg3[None, :], Wf, bf[None, :])
